# Initial kernel scaffold; baseline (speedup 1.0000x reference)
import jax
import jax.numpy as jnp
from jax.experimental import pallas as pl


def kernel(x, output_idx, process_indices, process_neurons, output_neurons):
    raise NotImplementedError("write your pallas kernel here")


# speedup vs baseline: 12.2070x; 12.2070x over previous
import jax
import jax.numpy as jnp
from jax.experimental import pallas as pl


def _body(x_ref, oidx_ref, pidx_ref, pn_ref, w_ref, out_ref):
    T, R = x_ref.shape
    NP = pn_ref.shape[0]
    NO = w_ref.shape[0] // R
    K = pidx_ref.shape[1]
    xt = x_ref[...]
    for i in range(K):
        idx = pidx_ref[:, i : i + 1]
        oh = (idx == jax.lax.broadcasted_iota(jnp.int32, (T, NP), 1)).astype(
            jnp.float32
        )
        v = jnp.dot(oh, pn_ref[...], preferred_element_type=jnp.float32)
        vns = jnp.sum(v * v, axis=1, keepdims=True) + 1e-8
        vtx = jnp.sum(xt * v, axis=1, keepdims=True)
        xt = xt - (2.0 * vtx / vns) * v
    ohe = (
        oidx_ref[...] == jax.lax.broadcasted_iota(jnp.int32, (T, NO), 1)
    ).astype(jnp.float32)
    xb = jnp.concatenate([xt * ohe[:, e : e + 1] for e in range(NO)], axis=1)
    out_ref[...] = jnp.dot(xb, w_ref[...], preferred_element_type=jnp.float32)


def kernel(x, output_idx, process_indices, process_neurons, output_neurons):
    B, S, R = x.shape
    NO, _, D = output_neurons.shape
    NP = process_neurons.shape[0]
    K = process_indices.shape[-1]
    xs = x.reshape(S, R)
    oidx = output_idx.reshape(S, 1)
    pidx = process_indices.reshape(S, K)
    wflat = output_neurons.reshape(NO * R, D)
    T = 256
    grid = (S // T,)
    out = pl.pallas_call(
        _body,
        grid=grid,
        in_specs=[
            pl.BlockSpec((T, R), lambda i: (i, 0)),
            pl.BlockSpec((T, 1), lambda i: (i, 0)),
            pl.BlockSpec((T, K), lambda i: (i, 0)),
            pl.BlockSpec((NP, R), lambda i: (0, 0)),
            pl.BlockSpec((NO * R, D), lambda i: (0, 0)),
        ],
        out_specs=pl.BlockSpec((T, D), lambda i: (i, 0)),
        out_shape=jax.ShapeDtypeStruct((S, D), jnp.float32),
    )(xs, oidx, pidx, process_neurons, wflat)
    return out.reshape(B, S, D)
